# baseline (speedup 1.0000x reference)
import functools

import jax
import jax.numpy as jnp
from jax import lax
from jax.experimental import pallas as pl
from jax.experimental.pallas import tpu as pltpu
from jax.experimental.pallas import tpu_sc as plsc

N = 10000
E = 320000
F = 128
HID = 128
NCLS = 40

NC = 2
NS = 16
NW = NC * NS
NPAD = 10112
RPT = NPAD // NS
BLK = 128
BPAD = 2560
WBLKS = BPAD // NW
JUNK = NPAD - 1

ROWBLK = 2000
NEG = -1e9

_MESH = plsc.VectorSubcoreMesh(core_axis_name="c", subcore_axis_name="s",
                               num_cores=NC, num_subcores=NS)

NBLKS = E // BLK
BASE_BLKS = NBLKS // NW
EXTRA = NBLKS - BASE_BLKS * NW


def _worker_blocks(cid, sid):
    wid = cid * NS + sid
    nblk = BASE_BLKS + jnp.where(wid < EXTRA, 1, 0)
    start = wid * BASE_BLKS + jnp.minimum(wid, EXTRA)
    return start, nblk


DEGW = 16


@functools.partial(
    pl.kernel,
    out_type=jax.ShapeDtypeStruct((NC, NPAD, DEGW), jnp.float32),
    mesh=_MESH,
    scratch_types=[
        pltpu.VMEM((BLK, DEGW), jnp.float32),
        pltpu.VMEM((BLK,), jnp.int32),
        pltpu.SemaphoreType.DMA,
        pltpu.VMEM_SHARED((NPAD, DEGW), jnp.float32),
    ],
)
def _deg_call(dst_hbm, out_hbm, buf, idx, dsem, accd):
    cid = lax.axis_index("c")
    sid = lax.axis_index("s")
    bbase, nblk = _worker_blocks(cid, sid)
    row0 = sid * RPT

    def fill(val):
        v16 = jnp.full((DEGW,), val, jnp.float32)

        def body(i, carry):
            buf[i, :] = v16
            return carry
        lax.fori_loop(0, BLK, body, 0)

    fill(0.0)
    for k in range(RPT // BLK):
        pltpu.sync_copy(buf, accd.at[pl.ds(row0 + k * BLK, BLK)])
    if RPT % BLK:
        pltpu.sync_copy(buf.at[pl.ds(0, RPT % BLK)],
                        accd.at[pl.ds(row0 + (RPT // BLK) * BLK, RPT % BLK)])
    plsc.subcore_barrier()
    fill(1.0)

    def fire(b, carry):
        pltpu.sync_copy(dst_hbm.at[bbase + b], idx)
        pltpu.sync_copy(buf, accd.at[idx], add=True)
        return carry
    lax.fori_loop(0, nblk, fire, 0)

    plsc.subcore_barrier()
    pltpu.sync_copy(accd.at[pl.ds(row0, RPT)],
                    out_hbm.at[cid, pl.ds(row0, RPT)])


RING = 3


@functools.partial(
    pl.kernel,
    out_type=jax.ShapeDtypeStruct((NC, NPAD, HID), jnp.float32),
    mesh=_MESH,
    scratch_types=[
        [pltpu.VMEM((2, BLK), jnp.int32) for _ in range(RING)],
        [pltpu.VMEM((BLK, HID), jnp.float32) for _ in range(RING)],
        [pltpu.SemaphoreType.DMA for _ in range(RING)],
        [pltpu.SemaphoreType.DMA for _ in range(RING)],
        [pltpu.SemaphoreType.DMA for _ in range(RING)],
        pltpu.VMEM_SHARED((NPAD, HID), jnp.float32),
    ],
)
def _agg_call(hs_hbm, sd_hbm, out_hbm, ibuf, rows, isem, gsem, ssem, acc):
    cid = lax.axis_index("c")
    sid = lax.axis_index("s")
    wid = cid * NS + sid
    blk0 = wid * 2 * WBLKS

    zero16 = jnp.zeros((16,), jnp.float32)

    def zero_rows(i, carry):
        for j in range(HID // 16):
            rows[0][i, pl.ds(j * 16, 16)] = zero16
        return carry
    lax.fori_loop(0, BLK, zero_rows, 0)
    row0 = sid * RPT
    for k in range(RPT // BLK):
        pltpu.sync_copy(rows[0], acc.at[pl.ds(row0 + k * BLK, BLK)])
    if RPT % BLK:
        pltpu.sync_copy(rows[0].at[pl.ds(0, RPT % BLK)],
                        acc.at[pl.ds(row0 + (RPT // BLK) * BLK, RPT % BLK)])
    plsc.subcore_barrier()

    def start_idx(b, s):
        pltpu.async_copy(sd_hbm.at[blk0 + 2 * b], ibuf[s].at[0], isem[s])
        pltpu.async_copy(sd_hbm.at[blk0 + 2 * b + 1], ibuf[s].at[1], isem[s])

    def wait_idx(b, s):
        pltpu.make_async_copy(sd_hbm.at[blk0 + 2 * b], ibuf[s].at[0],
                              isem[s]).wait()
        pltpu.make_async_copy(sd_hbm.at[blk0 + 2 * b + 1], ibuf[s].at[1],
                              isem[s]).wait()

    def start_gather(s):
        pltpu.async_copy(hs_hbm.at[ibuf[s].at[0]], rows[s], gsem[s])

    def wait_gather(s):
        pltpu.make_async_copy(hs_hbm.at[ibuf[s].at[0]], rows[s],
                              gsem[s]).wait()

    def start_scatter(s):
        pltpu.async_copy(rows[s], acc.at[ibuf[s].at[1]], ssem[s], add=True)

    def wait_scatter(s):
        pltpu.make_async_copy(rows[s], acc.at[ibuf[s].at[1]], ssem[s]).wait()

    pltpu.sync_copy(sd_hbm.at[blk0], ibuf[0].at[0])
    pltpu.sync_copy(sd_hbm.at[blk0 + 1], ibuf[0].at[1])
    start_gather(0)
    start_idx(1, 1)

    def step(b, u, drain_prev, launch_idx, launch_gather):
        nx = (u + 1) % RING
        ls = (u + 2) % RING
        if drain_prev:
            wait_scatter(ls)
        if launch_idx:
            start_idx(b + 2, ls)
        if launch_gather:
            wait_idx(b + 1, nx)
            start_gather(nx)
        wait_gather(u)
        start_scatter(u)

    def first_step():
        step(0, 0, False, True, True)

    def loop_body(i, carry):
        for u in range(RING):
            b = RING * i + u
            step(b, u, True, True, True)
        return carry

    first_step()
    step(1, 1, True, True, True)
    step(2, 2, True, True, True)
    lax.fori_loop(1, WBLKS // RING, loop_body, 0)
    step(78, 0, True, False, True)
    step(79, 1, True, False, False)
    wait_scatter(1)

    plsc.subcore_barrier()
    pltpu.sync_copy(acc.at[pl.ds(row0, RPT)], out_hbm.at[cid, pl.ds(row0, RPT)])


def _tc1_body(x_ref, w_ref, degp_ref, hs_ref, dinv_ref):
    deg = degp_ref[0, :, 0:1] + degp_ref[1, :, 0:1] + 1.0
    dinv = lax.rsqrt(deg)
    h = jnp.dot(x_ref[...], w_ref[...], preferred_element_type=jnp.float32)
    hs_ref[...] = h * dinv
    dinv_ref[...] = dinv


def _tc2_body(s_ref, hs_ref, dinv_ref, b_ref, w_ref, out_ref):
    dinv = dinv_ref[...]
    agg = s_ref[0] + s_ref[1] + hs_ref[...]
    x1 = jnp.maximum(dinv * agg + b_ref[...], 0.0)
    h2 = jnp.dot(x1, w_ref[...], preferred_element_type=jnp.float32)
    out_ref[...] = h2 * dinv


def _tc3_body(s_ref, hs_ref, dinv_ref, b_ref, w_ref, bfc_ref, out_ref):
    dinv = dinv_ref[...]
    agg = s_ref[0] + s_ref[1] + hs_ref[...]
    x2 = jnp.maximum(dinv * agg + b_ref[...], 0.0)
    logits = jnp.dot(x2, w_ref[...], preferred_element_type=jnp.float32)
    logits = logits + bfc_ref[...]
    m = jnp.max(logits, axis=1, keepdims=True)
    lse = jnp.log(jnp.sum(jnp.exp(logits - m), axis=1, keepdims=True)) + m
    out_ref[...] = logits - lse


_GRID = N // ROWBLK

_spec_rows = pl.BlockSpec((ROWBLK, HID), lambda i: (i, 0))
_spec_w = pl.BlockSpec((HID, HID), lambda i: (0, 0))
_spec_dinv = pl.BlockSpec((ROWBLK, 1), lambda i: (i, 0))
_spec_bias = pl.BlockSpec((1, HID), lambda i: (0, 0))
_spec_spart = pl.BlockSpec((NC, ROWBLK, HID), lambda i: (0, i, 0))
_spec_degp = pl.BlockSpec((NC, ROWBLK, DEGW), lambda i: (0, i, 0))

_tc1 = pl.pallas_call(
    _tc1_body,
    grid=(_GRID,),
    in_specs=[_spec_rows, _spec_w, _spec_degp],
    out_specs=[_spec_rows, _spec_dinv],
    out_shape=[
        jax.ShapeDtypeStruct((N, HID), jnp.float32),
        jax.ShapeDtypeStruct((N, 1), jnp.float32),
    ],
)

_tc2 = pl.pallas_call(
    _tc2_body,
    grid=(_GRID,),
    in_specs=[_spec_spart, _spec_rows, _spec_dinv, _spec_bias, _spec_w],
    out_specs=_spec_rows,
    out_shape=jax.ShapeDtypeStruct((N, HID), jnp.float32),
)

_tc3 = pl.pallas_call(
    _tc3_body,
    grid=(_GRID,),
    in_specs=[_spec_spart, _spec_rows, _spec_dinv, _spec_bias, _spec_w,
              _spec_bias],
    out_specs=_spec_rows,
    out_shape=jax.ShapeDtypeStruct((N, HID), jnp.float32),
)


def kernel(x, edge_index, W1, b1, W2, b2, Wfc, bfc):
    npad_e = BPAD * BLK - E
    srcp = jnp.concatenate(
        [edge_index[0], jnp.zeros((npad_e,), jnp.int32)]).reshape(BPAD, BLK)
    junk = N + jnp.arange(npad_e, dtype=jnp.int32) % (NPAD - N)
    dstp = jnp.concatenate([edge_index[1], junk]).reshape(BPAD, BLK)
    sd = jnp.stack([srcp, dstp], axis=1).reshape(2 * BPAD, BLK)

    dst2d = edge_index[1].reshape(NBLKS, BLK)
    degp = _deg_call(dst2d)

    hs1, dinv = _tc1(x, W1, degp)
    s1 = _agg_call(hs1, sd)
    hs2 = _tc2(s1, hs1, dinv, b1[None, :], W2)
    s2 = _agg_call(hs2, sd)

    wfc_pad = jnp.zeros((HID, HID), jnp.float32).at[:, :NCLS].set(Wfc)
    bfc_pad = jnp.full((1, HID), NEG, jnp.float32).at[0, :NCLS].set(bfc)
    out = _tc3(s2, hs2, dinv, b2[None, :], wfc_pad, bfc_pad)
    return out[:, :NCLS]


# speedup vs baseline: 2.7527x; 2.7527x over previous
import functools

import jax
import jax.numpy as jnp
from jax import lax
from jax.experimental import pallas as pl
from jax.experimental.pallas import tpu as pltpu
from jax.experimental.pallas import tpu_sc as plsc

N = 10000
E = 320000
F = 128
HID = 128
NCLS = 40

NC = 2
NS = 16
NW = NC * NS
NPAD = 10112
RPT = NPAD // NS
BLK = 128
BPAD = 2560
WBLKS = BPAD // NW
JUNK = NPAD - 1

ROWBLK = 2000
NEG = -1e9

_MESH = plsc.VectorSubcoreMesh(core_axis_name="c", subcore_axis_name="s",
                               num_cores=NC, num_subcores=NS)

NBLKS = E // BLK
BASE_BLKS = NBLKS // NW
EXTRA = NBLKS - BASE_BLKS * NW


def _worker_blocks(cid, sid):
    wid = cid * NS + sid
    nblk = BASE_BLKS + jnp.where(wid < EXTRA, 1, 0)
    start = wid * BASE_BLKS + jnp.minimum(wid, EXTRA)
    return start, nblk


DEGW = 16


@functools.partial(
    pl.kernel,
    out_type=jax.ShapeDtypeStruct((NC, NPAD, DEGW), jnp.float32),
    mesh=_MESH,
    scratch_types=[
        pltpu.VMEM((BLK, DEGW), jnp.float32),
        pltpu.VMEM((BLK,), jnp.int32),
        pltpu.SemaphoreType.DMA,
        pltpu.VMEM_SHARED((NPAD, DEGW), jnp.float32),
    ],
)
def _deg_call(dst_hbm, out_hbm, buf, idx, dsem, accd):
    cid = lax.axis_index("c")
    sid = lax.axis_index("s")
    bbase, nblk = _worker_blocks(cid, sid)
    row0 = sid * RPT

    def fill(val):
        v16 = jnp.full((DEGW,), val, jnp.float32)

        def body(i, carry):
            buf[i, :] = v16
            return carry
        lax.fori_loop(0, BLK, body, 0)

    fill(0.0)
    for k in range(RPT // BLK):
        pltpu.sync_copy(buf, accd.at[pl.ds(row0 + k * BLK, BLK)])
    if RPT % BLK:
        pltpu.sync_copy(buf.at[pl.ds(0, RPT % BLK)],
                        accd.at[pl.ds(row0 + (RPT // BLK) * BLK, RPT % BLK)])
    plsc.subcore_barrier()
    fill(1.0)

    def fire(b, carry):
        pltpu.sync_copy(dst_hbm.at[bbase + b], idx)
        pltpu.sync_copy(buf, accd.at[idx], add=True)
        return carry
    lax.fori_loop(0, nblk, fire, 0)

    plsc.subcore_barrier()
    pltpu.sync_copy(accd.at[pl.ds(row0, RPT)],
                    out_hbm.at[cid, pl.ds(row0, RPT)])


RING = 3
ABLKS = NBLKS // NW
LEFT0 = ABLKS * NW


@functools.partial(
    pl.kernel,
    out_type=jax.ShapeDtypeStruct((NC, NPAD, HID), jnp.float32),
    mesh=_MESH,
    scratch_types=[
        [pltpu.VMEM((BLK,), jnp.int32) for _ in range(RING)],
        [pltpu.VMEM((BLK,), jnp.int32) for _ in range(RING)],
        [pltpu.VMEM((BLK, HID), jnp.float32) for _ in range(RING)],
        [pltpu.SemaphoreType.DMA for _ in range(RING)],
        [pltpu.SemaphoreType.DMA for _ in range(RING)],
        pltpu.VMEM_SHARED((NPAD, HID), jnp.float32),
    ],
)
def _agg_call(hs_hbm, src_hbm, dst_hbm, out_hbm, sidx, didx, rows, gsem,
              ssem, acc):
    cid = lax.axis_index("c")
    sid = lax.axis_index("s")
    wid = cid * NS + sid
    blk0 = wid * ABLKS

    zero16 = jnp.zeros((16,), jnp.float32)

    def zero_rows(i, carry):
        for j in range(HID // 16):
            rows[0][i, pl.ds(j * 16, 16)] = zero16
        return carry
    lax.fori_loop(0, BLK, zero_rows, 0)
    row0 = sid * RPT
    for k in range(RPT // BLK):
        pltpu.sync_copy(rows[0], acc.at[pl.ds(row0 + k * BLK, BLK)])
    if RPT % BLK:
        pltpu.sync_copy(rows[0].at[pl.ds(0, RPT % BLK)],
                        acc.at[pl.ds(row0 + (RPT // BLK) * BLK, RPT % BLK)])
    plsc.subcore_barrier()

    def stage_idx(b, s):
        pltpu.sync_copy(src_hbm.at[b], sidx[s])
        pltpu.sync_copy(dst_hbm.at[b], didx[s])

    def start_gather(s):
        pltpu.async_copy(hs_hbm.at[sidx[s]], rows[s], gsem[s])

    def wait_gather(s):
        pltpu.make_async_copy(hs_hbm.at[sidx[s]], rows[s], gsem[s]).wait()

    def start_scatter(s):
        pltpu.async_copy(rows[s], acc.at[didx[s]], ssem[s], add=True)

    def wait_scatter(s):
        pltpu.make_async_copy(rows[s], acc.at[didx[s]], ssem[s]).wait()

    stage_idx(blk0, 0)
    start_gather(0)
    stage_idx(blk0 + 1, 1)
    start_gather(1)

    def body(i, carry):
        for u in range(RING):
            b = RING * i + u
            ns = (u + 2) % RING

            @pl.when(jnp.logical_and(b + 2 < ABLKS, b >= 1))
            def _drain():
                wait_scatter(ns)

            @pl.when(b + 2 < ABLKS)
            def _launch():
                stage_idx(blk0 + b + 2, ns)
                start_gather(ns)

            wait_gather(u)
            start_scatter(u)
        return carry
    lax.fori_loop(0, ABLKS // RING, body, 0)

    for s in range(RING):
        wait_scatter(s)

    @pl.when(wid < NBLKS - LEFT0)
    def _leftover():
        stage_idx(LEFT0 + wid, 0)
        start_gather(0)
        wait_gather(0)
        start_scatter(0)
        wait_scatter(0)

    plsc.subcore_barrier()
    pltpu.sync_copy(acc.at[pl.ds(row0, RPT)], out_hbm.at[cid, pl.ds(row0, RPT)])


def _tc1_body(x_ref, w_ref, degp_ref, hs_ref, dinv_ref):
    deg = degp_ref[0, :, 0:1] + degp_ref[1, :, 0:1] + 1.0
    dinv = lax.rsqrt(deg)
    h = jnp.dot(x_ref[...], w_ref[...], preferred_element_type=jnp.float32)
    hs_ref[...] = h * dinv
    dinv_ref[...] = dinv


def _tc2_body(s_ref, hs_ref, dinv_ref, b_ref, w_ref, out_ref):
    dinv = dinv_ref[...]
    agg = s_ref[0] + s_ref[1] + hs_ref[...]
    x1 = jnp.maximum(dinv * agg + b_ref[...], 0.0)
    h2 = jnp.dot(x1, w_ref[...], preferred_element_type=jnp.float32)
    out_ref[...] = h2 * dinv


def _tc3_body(s_ref, hs_ref, dinv_ref, b_ref, w_ref, bfc_ref, out_ref):
    dinv = dinv_ref[...]
    agg = s_ref[0] + s_ref[1] + hs_ref[...]
    x2 = jnp.maximum(dinv * agg + b_ref[...], 0.0)
    logits = jnp.dot(x2, w_ref[...], preferred_element_type=jnp.float32)
    logits = logits + bfc_ref[...]
    m = jnp.max(logits, axis=1, keepdims=True)
    lse = jnp.log(jnp.sum(jnp.exp(logits - m), axis=1, keepdims=True)) + m
    out_ref[...] = logits - lse


_GRID = N // ROWBLK

_spec_rows = pl.BlockSpec((ROWBLK, HID), lambda i: (i, 0))
_spec_w = pl.BlockSpec((HID, HID), lambda i: (0, 0))
_spec_dinv = pl.BlockSpec((ROWBLK, 1), lambda i: (i, 0))
_spec_bias = pl.BlockSpec((1, HID), lambda i: (0, 0))
_spec_spart = pl.BlockSpec((NC, ROWBLK, HID), lambda i: (0, i, 0))
_spec_degp = pl.BlockSpec((NC, ROWBLK, DEGW), lambda i: (0, i, 0))

_tc1 = pl.pallas_call(
    _tc1_body,
    grid=(_GRID,),
    in_specs=[_spec_rows, _spec_w, _spec_degp],
    out_specs=[_spec_rows, _spec_dinv],
    out_shape=[
        jax.ShapeDtypeStruct((N, HID), jnp.float32),
        jax.ShapeDtypeStruct((N, 1), jnp.float32),
    ],
)

_tc2 = pl.pallas_call(
    _tc2_body,
    grid=(_GRID,),
    in_specs=[_spec_spart, _spec_rows, _spec_dinv, _spec_bias, _spec_w],
    out_specs=_spec_rows,
    out_shape=jax.ShapeDtypeStruct((N, HID), jnp.float32),
)

_tc3 = pl.pallas_call(
    _tc3_body,
    grid=(_GRID,),
    in_specs=[_spec_spart, _spec_rows, _spec_dinv, _spec_bias, _spec_w,
              _spec_bias],
    out_specs=_spec_rows,
    out_shape=jax.ShapeDtypeStruct((N, HID), jnp.float32),
)


def kernel(x, edge_index, W1, b1, W2, b2, Wfc, bfc):
    src2d = edge_index[0].reshape(NBLKS, BLK)
    dst2d = edge_index[1].reshape(NBLKS, BLK)
    degp = _deg_call(dst2d)

    hs1, dinv = _tc1(x, W1, degp)
    s1 = _agg_call(hs1, src2d, dst2d)
    hs2 = _tc2(s1, hs1, dinv, b1[None, :], W2)
    s2 = _agg_call(hs2, src2d, dst2d)

    wfc_pad = jnp.zeros((HID, HID), jnp.float32).at[:, :NCLS].set(Wfc)
    bfc_pad = jnp.full((1, HID), NEG, jnp.float32).at[0, :NCLS].set(bfc)
    out = _tc3(s2, hs2, dinv, b2[None, :], wfc_pad, bfc_pad)
    return out[:, :NCLS]
